# baseline (speedup 1.0000x reference)
import functools

import jax
import jax.numpy as jnp
import numpy as np
from jax import lax
from jax.experimental import pallas as pl
from jax.experimental.pallas import tpu as pltpu
from jax.experimental.pallas import tpu_sc as plsc

B = 16384
F = 26
FP = 32
D = 16
NS_GRP = 4
N_FLAT = B * FP
OUT_ROWS = NS_GRP * B
HID1, HID2 = 256, 128
IN_DIM = F * D
PAD_DIM = FP * D
TOTAL_ROWS = F * 100000

NC, NS = 2, 16
NW = NC * NS
PER_W = N_FLAT // NW
CHUNK = 2048
NCHUNK = PER_W // CHUNK
G = CHUNK // 128

_eff = np.arange(FP, dtype=np.int64)
_eff = np.where(_eff < F, _eff, _eff - F)
_OFF_PATTERN = (_eff * 100000).astype(np.int32)

SAMP_PER_CHUNK = CHUNK // FP
SAMP_PER_W = PER_W // FP


def _gather_body(idx_hbm, offp_hbm, table_hbm, out_hbm,
                 idx2_v, idx_v, offp_v, rows_v, wide_v, sem):
    wid = lax.axis_index("s") * NC + lax.axis_index("c")
    pltpu.sync_copy(offp_hbm, offp_v)

    def chunk_body(c, _):
        b0 = wid * SAMP_PER_W + c * SAMP_PER_CHUNK
        pltpu.sync_copy(idx_hbm.at[pl.ds(b0, SAMP_PER_CHUNK)], idx2_v)

        def flat_body(r, _):
            for c2 in (0, 16):
                s = pl.ds(c2, 16)
                idx_v[pl.ds(r * FP + c2, 16)] = idx2_v[r, s] + offp_v[s]
            return 0

        lax.fori_loop(0, SAMP_PER_CHUNK, flat_body, 0)
        copies = []
        for g in range(G):
            copies.append(pltpu.async_copy(
                table_hbm.at[idx_v.at[pl.ds(g * 128, 128)]],
                rows_v.at[pl.ds(g * 128, 128)],
                sem))
        for cp in copies:
            cp.wait()

        def bounce_body(r, _):
            s = r // SAMP_PER_CHUNK
            b = r % SAMP_PER_CHUNK
            k = b * FP + s * 8
            for v in range(8):
                wide_v[r, pl.ds(v * D, D)] = rows_v[k + v, :]
            return 0

        lax.fori_loop(0, CHUNK // 8, bounce_body, 0)
        for s in range(NS_GRP):
            pltpu.sync_copy(
                wide_v.at[pl.ds(s * SAMP_PER_CHUNK, SAMP_PER_CHUNK)],
                out_hbm.at[pl.ds(s * B + b0, SAMP_PER_CHUNK)])
        return 0

    lax.fori_loop(0, NCHUNK, chunk_body, 0)


def _sc_gather(idx2d, table):
    mesh = plsc.VectorSubcoreMesh(core_axis_name="c", subcore_axis_name="s")
    k = functools.partial(
        pl.kernel,
        mesh=mesh,
        compiler_params=pltpu.CompilerParams(use_tc_tiling_on_sc=False),
        out_type=jax.ShapeDtypeStruct((OUT_ROWS, 128), jnp.float32),
        scratch_types=[
            pltpu.VMEM((SAMP_PER_CHUNK, FP), jnp.int32),
            pltpu.VMEM((CHUNK,), jnp.int32),
            pltpu.VMEM((FP,), jnp.int32),
            pltpu.VMEM((CHUNK, D), jnp.float32),
            pltpu.VMEM((CHUNK // 8, 128), jnp.float32),
            pltpu.SemaphoreType.DMA,
        ],
    )(_gather_body)
    return k(idx2d, jnp.asarray(_OFF_PATTERN), table)


def _mlp_body(h0_ref, h1_ref, h2_ref, h3_ref,
              w1_ref, b1_ref, w2_ref, b2_ref, wo_ref, bo_ref, o_ref):
    planes = (h0_ref, h1_ref, h2_ref, h3_ref)
    blk = h0_ref.shape[0]
    acc = jnp.zeros((blk, HID1), jnp.float32)
    for s in range(NS_GRP):
        acc += jnp.dot(planes[s][...], w1_ref[pl.ds(s * 128, 128), :],
                       preferred_element_type=jnp.float32)
    h1 = jnp.maximum(acc + b1_ref[...], 0.0)
    h2 = jnp.maximum(
        jnp.dot(h1, w2_ref[...], preferred_element_type=jnp.float32)
        + b2_ref[...], 0.0)
    logit = jnp.dot(h2, wo_ref[...],
                    preferred_element_type=jnp.float32)[:, 0] + bo_ref[...]
    o_ref[...] = jax.nn.sigmoid(logit)


def _tc_mlp(planes, W1p, b1, W2, b2, Wo, bo):
    BLK = 2048
    nblk = B // BLK
    in_specs = [
        pl.BlockSpec((BLK, 128), lambda i, s=s: (s * nblk + i, 0))
        for s in range(NS_GRP)
    ] + [
        pl.BlockSpec((PAD_DIM, HID1), lambda i: (0, 0)),
        pl.BlockSpec((HID1,), lambda i: (0,)),
        pl.BlockSpec((HID1, HID2), lambda i: (0, 0)),
        pl.BlockSpec((HID2,), lambda i: (0,)),
        pl.BlockSpec((HID2, 1), lambda i: (0, 0)),
        pl.BlockSpec((1,), lambda i: (0,)),
    ]
    return pl.pallas_call(
        _mlp_body,
        grid=(nblk,),
        in_specs=in_specs,
        out_specs=pl.BlockSpec((BLK,), lambda i: (i,)),
        out_shape=jax.ShapeDtypeStruct((B,), jnp.float32),
    )(planes, planes, planes, planes, W1p, b1, W2, b2, Wo, bo)


def kernel(x, table, W1, b1, W2, b2, Wo, bo):
    xi = x.astype(jnp.int32)
    xp = jnp.concatenate([xi, xi[:, :FP - F]], axis=1)
    table_lin = jnp.concatenate([table[:, D // 2:], table[:, :D // 2]],
                                axis=1)
    planes = _sc_gather(xp, table_lin)
    W1p = jnp.concatenate(
        [W1, jnp.zeros((PAD_DIM - IN_DIM, HID1), jnp.float32)], axis=0)
    w1g = W1p.reshape(FP, D, HID1)
    W1p = jnp.concatenate([w1g[:, D // 2:, :], w1g[:, :D // 2, :]],
                          axis=1).reshape(PAD_DIM, HID1)
    return _tc_mlp(planes, W1p, b1, W2, b2, Wo, bo)


# speedup vs baseline: 1.2026x; 1.2026x over previous
import functools

import jax
import jax.numpy as jnp
import numpy as np
from jax import lax
from jax.experimental import pallas as pl
from jax.experimental.pallas import tpu as pltpu
from jax.experimental.pallas import tpu_sc as plsc

B = 16384
F = 26
FP = 32
D = 16
NS_GRP = 4
N_FLAT = B * FP
OUT_ROWS = NS_GRP * B
HID1, HID2 = 256, 128
IN_DIM = F * D
PAD_DIM = FP * D
TOTAL_ROWS = F * 100000

NC, NS = 2, 16
NW = NC * NS
PER_W = N_FLAT // NW
CHUNK = 2048
NCHUNK = PER_W // CHUNK
G = CHUNK // 128

_eff = np.arange(FP, dtype=np.int64)
_eff = np.where(_eff < F, _eff, _eff - F)
_OFF_PLANES = np.stack([
    (_eff[8 * s + (np.arange(CHUNK) % 8)] * 100000).astype(np.int32)
    for s in range(NS_GRP)
]).reshape(-1)


def _gather_body(idx_hbm, offp_hbm, table_hbm, out_hbm,
                 idx_v, offp_v, rows_v, wide_v, sem):
    wid = lax.axis_index("s") * NC + lax.axis_index("c")
    plane = wid // (NW // NS_GRP)
    pltpu.sync_copy(offp_hbm.at[pl.ds(plane * CHUNK, CHUNK)], offp_v)

    def chunk_body(c, _):
        base = wid * PER_W + c * CHUNK
        pltpu.sync_copy(idx_hbm.at[pl.ds(base, CHUNK)], idx_v)
        for j in range(CHUNK // 16):
            s = pl.ds(j * 16, 16)
            idx_v[s] = idx_v[s] + offp_v[s]
        copies = []
        for g in range(G):
            copies.append(pltpu.async_copy(
                table_hbm.at[idx_v.at[pl.ds(g * 128, 128)]],
                rows_v.at[pl.ds(g * 128, 128)],
                sem))
        for cp in copies:
            cp.wait()

        def bounce_body(r, _):
            for v in range(8):
                wide_v[r, pl.ds(v * D, D)] = rows_v[8 * r + v, :]
            return 0

        lax.fori_loop(0, CHUNK // 8, bounce_body, 0)
        pltpu.sync_copy(wide_v, out_hbm.at[pl.ds(base // 8, CHUNK // 8)])
        return 0

    lax.fori_loop(0, NCHUNK, chunk_body, 0)


def _sc_gather(idx_flat, table):
    mesh = plsc.VectorSubcoreMesh(core_axis_name="c", subcore_axis_name="s")
    k = functools.partial(
        pl.kernel,
        mesh=mesh,
        compiler_params=pltpu.CompilerParams(use_tc_tiling_on_sc=False),
        out_type=jax.ShapeDtypeStruct((OUT_ROWS, 128), jnp.float32),
        scratch_types=[
            pltpu.VMEM((CHUNK,), jnp.int32),
            pltpu.VMEM((CHUNK,), jnp.int32),
            pltpu.VMEM((CHUNK, D), jnp.float32),
            pltpu.VMEM((CHUNK // 8, 128), jnp.float32),
            pltpu.SemaphoreType.DMA,
        ],
    )(_gather_body)
    return k(idx_flat, jnp.asarray(_OFF_PLANES), table)


def _mlp_body(h0_ref, h1_ref, h2_ref, h3_ref,
              w1_ref, b1_ref, w2_ref, b2_ref, wo_ref, bo_ref, o_ref):
    planes = (h0_ref, h1_ref, h2_ref, h3_ref)
    blk = h0_ref.shape[0]
    acc = jnp.zeros((blk, HID1), jnp.float32)
    for s in range(NS_GRP):
        acc += jnp.dot(planes[s][...], w1_ref[pl.ds(s * 128, 128), :],
                       preferred_element_type=jnp.float32)
    h1 = jnp.maximum(acc + b1_ref[...], 0.0)
    h2 = jnp.maximum(
        jnp.dot(h1, w2_ref[...], preferred_element_type=jnp.float32)
        + b2_ref[...], 0.0)
    logit = jnp.dot(h2, wo_ref[...],
                    preferred_element_type=jnp.float32)[:, 0] + bo_ref[...]
    o_ref[...] = jax.nn.sigmoid(logit)


def _tc_mlp(planes, W1p, b1, W2, b2, Wo, bo):
    BLK = 2048
    nblk = B // BLK
    in_specs = [
        pl.BlockSpec((BLK, 128), lambda i, s=s: (s * nblk + i, 0))
        for s in range(NS_GRP)
    ] + [
        pl.BlockSpec((PAD_DIM, HID1), lambda i: (0, 0)),
        pl.BlockSpec((HID1,), lambda i: (0,)),
        pl.BlockSpec((HID1, HID2), lambda i: (0, 0)),
        pl.BlockSpec((HID2,), lambda i: (0,)),
        pl.BlockSpec((HID2, 1), lambda i: (0, 0)),
        pl.BlockSpec((1,), lambda i: (0,)),
    ]
    return pl.pallas_call(
        _mlp_body,
        grid=(nblk,),
        in_specs=in_specs,
        out_specs=pl.BlockSpec((BLK,), lambda i: (i,)),
        out_shape=jax.ShapeDtypeStruct((B,), jnp.float32),
    )(planes, planes, planes, planes, W1p, b1, W2, b2, Wo, bo)


def kernel(x, table, W1, b1, W2, b2, Wo, bo):
    xi = x.astype(jnp.int32)
    xp = jnp.concatenate([xi, xi[:, :FP - F]], axis=1)
    idx_flat = xp.reshape(B, NS_GRP, 8).transpose(1, 0, 2).reshape(N_FLAT)
    planes = _sc_gather(idx_flat, table)
    W1p = jnp.concatenate(
        [W1, jnp.zeros((PAD_DIM - IN_DIM, HID1), jnp.float32)], axis=0)
    return _tc_mlp(planes, W1p, b1, W2, b2, Wo, bo)
